# Initial kernel scaffold; baseline (speedup 1.0000x reference)
import jax
import jax.numpy as jnp
from jax.experimental import pallas as pl


def kernel(x, gold):
    raise NotImplementedError("write your pallas kernel here")


# speedup vs baseline: 2.6904x; 2.6904x over previous
import functools
import math

import jax
import jax.numpy as jnp
from jax import lax
from jax.experimental import pallas as pl
from jax.experimental.pallas import tpu as pltpu
from jax.experimental.pallas import tpu_sc as plsc

_SIZE = 32768
_N_TOK = 4096
_SMOOTH = 0.1
_CONF = 1.0 - _SMOOTH
_SV = _SMOOTH / (_SIZE - 1)
_CONST = (_SIZE - 1) * _SV * math.log(_SV) + _CONF * math.log(_CONF)
_DELTA = _CONF - _SV
_PAD_VAL = -100

_L = 16
_NW = 32
_BPW = _N_TOK // _NW
_NCH = _BPW // _L
_ROW_W = _SIZE // _L


def _sc_gather(x1d, gold_flat):
    mesh = plsc.VectorSubcoreMesh(core_axis_name="c", subcore_axis_name="s")

    @functools.partial(
        pl.kernel,
        mesh=mesh,
        out_type=jax.ShapeDtypeStruct((_NW, _L), jnp.float32),
        scratch_types=[
            pltpu.VMEM((_BPW,), jnp.int32),
            pltpu.VMEM((_BPW,), jnp.int32),
            pltpu.VMEM((_BPW,), jnp.float32),
            pltpu.VMEM((_L,), jnp.float32),
            pltpu.SemaphoreType.DMA,
        ],
    )
    def k(x_hbm, gold_hbm, out_hbm, gold_v, idx_v, val_v, acc_v, sem):
        wid = lax.axis_index("s") * 2 + lax.axis_index("c")
        base = wid * _BPW
        pltpu.sync_copy(gold_hbm.at[pl.ds(base, _BPW)], gold_v)
        iota = lax.iota(jnp.int32, _L)
        for j in range(_NCH):
            g = gold_v[pl.ds(j * _L, _L)]
            g = jnp.where(g == _PAD_VAL, 0, g)
            tok = base + j * _L + iota
            idx_v[pl.ds(j * _L, _L)] = tok * _SIZE + g
        pltpu.async_copy(x_hbm.at[idx_v], val_v, sem).wait()
        acc = jnp.zeros((_L,), jnp.float32)
        for j in range(_NCH):
            acc = acc + val_v[pl.ds(j * _L, _L)]
        acc_v[...] = acc
        pltpu.sync_copy(acc_v, out_hbm.at[wid])

    return k(x1d, gold_flat)


_BR = 128
_GRID = _N_TOK // _BR


def _tc_reduce(x, partials):
    def body(x_ref, p_ref, out_ref, acc_ref):
        i = pl.program_id(0)

        @pl.when(i == 0)
        def _():
            acc_ref[0] = 0.0

        acc_ref[0] += jnp.sum(x_ref[...])

        @pl.when(i == _GRID - 1)
        def _():
            s_gold = jnp.sum(p_ref[...])
            out_ref[0, 0] = _CONST - (
                _SV * acc_ref[0] + _DELTA * s_gold) / _N_TOK

    return pl.pallas_call(
        body,
        grid=(_GRID,),
        in_specs=[
            pl.BlockSpec((_BR, _SIZE), lambda i: (i, 0)),
            pl.BlockSpec((_NW, _L), lambda i: (0, 0)),
        ],
        out_specs=pl.BlockSpec(memory_space=pltpu.SMEM),
        out_shape=jax.ShapeDtypeStruct((1, 1), jnp.float32),
        scratch_shapes=[pltpu.SMEM((1,), jnp.float32)],
    )(x, partials)


def kernel(x, gold):
    x1d = x.reshape(_N_TOK * _SIZE)
    gold_flat = gold.reshape(-1)
    partials = _sc_gather(x1d, gold_flat)
    return _tc_reduce(x, partials)[0, 0]


# speedup vs baseline: 7.3775x; 2.7421x over previous
import functools
import math

import jax
import jax.numpy as jnp
from jax import lax
from jax.experimental import pallas as pl
from jax.experimental.pallas import tpu as pltpu
from jax.experimental.pallas import tpu_sc as plsc

_SIZE = 32768
_N_TOK = 4096
_SMOOTH = 0.1
_CONF = 1.0 - _SMOOTH
_SV = _SMOOTH / (_SIZE - 1)
_CONST = (_SIZE - 1) * _SV * math.log(_SV) + _CONF * math.log(_CONF)
_DELTA = _CONF - _SV
_PAD_VAL = -100

_L = 16
_NW = 32
_BPW = _N_TOK // _NW
_NCH = _BPW // _L
_ROW_W = _SIZE // _L


def _sc_gather(x1d, gold_flat):
    mesh = plsc.VectorSubcoreMesh(core_axis_name="c", subcore_axis_name="s")

    @functools.partial(
        pl.kernel,
        mesh=mesh,
        out_type=jax.ShapeDtypeStruct((_NW, _L), jnp.float32),
        scratch_types=[
            pltpu.VMEM((_BPW,), jnp.int32),
            pltpu.VMEM((_L, 8, 128), jnp.float32),
            pltpu.VMEM((_L,), jnp.float32),
            pltpu.SemaphoreType.DMA,
        ],
    )
    def k(x_hbm, gold_hbm, out_hbm, gold_v, tiles_v, acc_v, sem):
        wid = lax.axis_index("s") * 2 + lax.axis_index("c")
        base = wid * _BPW
        pltpu.sync_copy(gold_hbm.at[pl.ds(base, _BPW)], gold_v)
        iota = lax.iota(jnp.int32, _L)
        acc = jnp.zeros((_L,), jnp.float32)
        for j in range(_NCH):
            gvec = gold_v[pl.ds(j * _L, _L)]
            gvec = jnp.where(gvec == _PAD_VAL, 0, gvec)
            handles = []
            for i in range(_L):
                t = j * _L + i
                g = gvec[i]
                cb = pl.multiple_of(jnp.bitwise_and(g, -128), 128)
                rb = pl.multiple_of(base + (t & ~7), 8)
                handles.append(pltpu.make_async_copy(
                    x_hbm.at[pl.ds(rb, 8), pl.ds(cb, 128)],
                    tiles_v.at[i], sem))
            for h in handles:
                h.start()
            for h in handles:
                h.wait()
            lanes = gvec & (_L - 1)
            vals = jnp.zeros((_L,), jnp.float32)
            for i in range(_L):
                t = j * _L + i
                g = gvec[i]
                cb16 = jnp.bitwise_and(jnp.bitwise_and(g, 127), -16)
                v_i = tiles_v[i, t & 7, pl.ds(cb16, _L)]
                picked = v_i.at[lanes].get(mode="promise_in_bounds")
                vals = jnp.where(iota == i, picked, vals)
            acc = acc + vals
        acc_v[...] = acc
        pltpu.sync_copy(acc_v, out_hbm.at[wid])

    return k(x1d, gold_flat)


_BR = 128
_GRID = _N_TOK // _BR


def _tc_reduce(x, partials):
    def body(x_ref, p_ref, out_ref, acc_ref):
        i = pl.program_id(0)

        @pl.when(i == 0)
        def _():
            acc_ref[0] = 0.0

        acc_ref[0] += jnp.sum(x_ref[...])

        @pl.when(i == _GRID - 1)
        def _():
            s_gold = jnp.sum(p_ref[...])
            out_ref[0, 0] = _CONST - (
                _SV * acc_ref[0] + _DELTA * s_gold) / _N_TOK

    return pl.pallas_call(
        body,
        grid=(_GRID,),
        in_specs=[
            pl.BlockSpec((_BR, _SIZE), lambda i: (i, 0)),
            pl.BlockSpec((_NW, _L), lambda i: (0, 0)),
        ],
        out_specs=pl.BlockSpec(memory_space=pltpu.SMEM),
        out_shape=jax.ShapeDtypeStruct((1, 1), jnp.float32),
        scratch_shapes=[pltpu.SMEM((1,), jnp.float32)],
    )(x, partials)


def kernel(x, gold):
    gold_flat = gold.reshape(-1)
    partials = _sc_gather(x, gold_flat)
    return _tc_reduce(x, partials)[0, 0]
